# baseline (speedup 1.0000x reference)
import functools

import jax
import jax.numpy as jnp
from jax import lax
from jax.experimental import pallas as pl
from jax.experimental.pallas import tpu as pltpu
from jax.experimental.pallas import tpu_sc as plsc

_BATCH = 16384
_NCLS = 392
_NC = 2
_NS = 16
_NW = _NC * _NS
_L = 16
_ROWS_PER_W = _BATCH // _NW
_CHUNK = 64
_NCHUNK = _ROWS_PER_W // _CHUNK
_GPC = _CHUNK // _L
_CHUNK_ELEMS = _CHUNK * _NCLS

_mesh = plsc.VectorSubcoreMesh(core_axis_name="c", subcore_axis_name="s")


@functools.partial(
    pl.kernel,
    mesh=_mesh,
    out_type=jax.ShapeDtypeStruct((_NW, _L), jnp.float32),
    scratch_types=[
        pltpu.VMEM((_CHUNK, _NCLS), jnp.float32),
        pltpu.VMEM((_CHUNK, _NCLS), jnp.float32),
        pltpu.VMEM((_ROWS_PER_W,), jnp.int32),
        pltpu.VMEM((_L,), jnp.float32),
        pltpu.SemaphoreType.DMA,
        pltpu.SemaphoreType.DMA,
    ],
    compiler_params=pltpu.CompilerParams(
        use_tc_tiling_on_sc=True, needs_layout_passes=False),
)
def _iou_partials(prob_hbm, label_hbm, out_hbm,
                  buf_a, buf_b, lab_v, acc_v, sem_a, sem_b):
    wid = lax.axis_index("s") * _NC + lax.axis_index("c")
    row0 = wid * _ROWS_PER_W
    pltpu.sync_copy(label_hbm.at[pl.ds(row0, _ROWS_PER_W)], lab_v)

    iota = lax.iota(jnp.int32, _L)
    row_base = iota * _NCLS
    neg_inf = jnp.full((_L,), -jnp.inf, jnp.float32)

    def chunk_copy(c, buf, sem):
        return pltpu.make_async_copy(
            prob_hbm.at[pl.ds(row0 + c * _CHUNK, _CHUNK)], buf, sem)

    def process(buf, chunk, acc):
        def group_body(gi, acc):
            rows = gi * _L + iota

            def j_body(j, ms):
                m0, m1, m2, m3, m4, m5 = ms
                x = plsc.load_gather(buf, [rows, jnp.full((_L,), j, jnp.int32)])
                n0 = jnp.maximum(m0, x)
                c = jnp.minimum(m0, x)
                n1 = jnp.maximum(m1, c)
                c = jnp.minimum(m1, c)
                n2 = jnp.maximum(m2, c)
                c = jnp.minimum(m2, c)
                n3 = jnp.maximum(m3, c)
                c = jnp.minimum(m3, c)
                n4 = jnp.maximum(m4, c)
                c = jnp.minimum(m4, c)
                n5 = jnp.maximum(m5, c)
                return (n0, n1, n2, n3, n4, n5)

            m0, m1, m2, m3, m4, m5 = lax.fori_loop(
                0, _NCLS, j_body, (neg_inf,) * 6, unroll=8)

            t = m5
            cnt = ((m0 > t).astype(jnp.float32)
                   + (m1 > t).astype(jnp.float32)
                   + (m2 > t).astype(jnp.float32)
                   + (m3 > t).astype(jnp.float32)
                   + (m4 > t).astype(jnp.float32))
            lab16 = plsc.load_gather(
                lab_v, [(chunk * _GPC + gi) * _L + iota])
            labval = plsc.load_gather(buf, [rows, lab16])
            inter = (labval > t).astype(jnp.float32)
            union = cnt + 1.0 - inter
            return acc + inter / union

        return lax.fori_loop(0, _GPC, group_body, acc)

    chunk_copy(0, buf_a, sem_a).start()

    def pair_body(i, acc):
        ca = 2 * i
        cb = 2 * i + 1
        chunk_copy(ca, buf_a, sem_a).wait()
        chunk_copy(cb, buf_b, sem_b).start()
        acc = process(buf_a, ca, acc)
        chunk_copy(cb, buf_b, sem_b).wait()
        chunk_copy(jnp.minimum(ca + 2, _NCHUNK - 1), buf_a, sem_a).start()
        acc = process(buf_b, cb, acc)
        return acc

    acc = lax.fori_loop(0, _NCHUNK // 2, pair_body,
                        jnp.zeros((_L,), jnp.float32))
    chunk_copy(_NCHUNK - 1, buf_a, sem_a).wait()
    acc_v[...] = acc
    pltpu.sync_copy(acc_v, out_hbm.at[wid])


def kernel(prob, label):
    partials = _iou_partials(prob, label)
    return jnp.sum(partials) / jnp.float32(_BATCH)


# speedup vs baseline: 2.5134x; 2.5134x over previous
import functools

import jax
import jax.numpy as jnp
from jax import lax
from jax.experimental import pallas as pl
from jax.experimental.pallas import tpu as pltpu
from jax.experimental.pallas import tpu_sc as plsc

_BATCH = 16384
_NCLS = 392
_NC = 2
_NS = 16
_NW = _NC * _NS
_L = 16
_ROWS_PER_W = _BATCH // _NW
_CHUNK = 128
_NCHUNK = _ROWS_PER_W // _CHUNK
_GPC = _CHUNK // _L

_mesh = plsc.VectorSubcoreMesh(core_axis_name="c", subcore_axis_name="s")


@functools.partial(
    pl.kernel,
    mesh=_mesh,
    out_type=jax.ShapeDtypeStruct((_NW, _L), jnp.float32),
    scratch_types=[
        pltpu.VMEM((_NCLS, _CHUNK), jnp.float32),
        pltpu.VMEM((_NCLS, _CHUNK), jnp.float32),
        pltpu.VMEM((_ROWS_PER_W,), jnp.int32),
        pltpu.VMEM((_L,), jnp.float32),
        pltpu.SemaphoreType.DMA,
        pltpu.SemaphoreType.DMA,
    ],
    compiler_params=pltpu.CompilerParams(
        use_tc_tiling_on_sc=True, needs_layout_passes=False),
)
def _iou_partials(probt_hbm, label_hbm, out_hbm,
                  buf_a, buf_b, lab_v, acc_v, sem_a, sem_b):
    wid = lax.axis_index("s") * _NC + lax.axis_index("c")
    col0 = wid * _ROWS_PER_W
    pltpu.sync_copy(label_hbm.at[pl.ds(col0, _ROWS_PER_W)], lab_v)

    iota = lax.iota(jnp.int32, _L)
    neg_inf = jnp.full((_L,), -jnp.inf, jnp.float32)

    def chunk_copy(c, buf, sem):
        return pltpu.make_async_copy(
            probt_hbm.at[:, pl.ds(col0 + c * _CHUNK, _CHUNK)], buf, sem)

    def process(buf, chunk, acc):
        for g in range(_GPC):
            gcol = g * _L

            def j_body(j, ms):
                m0, m1, m2, m3, m4, m5 = ms
                x = buf[j, pl.ds(gcol, _L)]
                n0 = jnp.maximum(m0, x)
                c = jnp.minimum(m0, x)
                n1 = jnp.maximum(m1, c)
                c = jnp.minimum(m1, c)
                n2 = jnp.maximum(m2, c)
                c = jnp.minimum(m2, c)
                n3 = jnp.maximum(m3, c)
                c = jnp.minimum(m3, c)
                n4 = jnp.maximum(m4, c)
                c = jnp.minimum(m4, c)
                n5 = jnp.maximum(m5, c)
                return (n0, n1, n2, n3, n4, n5)

            m0, m1, m2, m3, m4, m5 = lax.fori_loop(
                0, _NCLS, j_body, (neg_inf,) * 6, unroll=8)

            t = m5
            cnt = ((m0 > t).astype(jnp.float32)
                   + (m1 > t).astype(jnp.float32)
                   + (m2 > t).astype(jnp.float32)
                   + (m3 > t).astype(jnp.float32)
                   + (m4 > t).astype(jnp.float32))
            lab16 = plsc.load_gather(
                lab_v, [(chunk * _GPC + g) * _L + iota])
            labval = plsc.load_gather(buf, [lab16, gcol + iota])
            inter = (labval > t).astype(jnp.float32)
            union = cnt + 1.0 - inter
            acc = acc + inter / union
        return acc

    acc = jnp.zeros((_L,), jnp.float32)
    chunk_copy(0, buf_a, sem_a).start()
    chunk_copy(1, buf_b, sem_b).start()
    chunk_copy(0, buf_a, sem_a).wait()
    acc = process(buf_a, 0, acc)
    chunk_copy(2, buf_a, sem_a).start()
    chunk_copy(1, buf_b, sem_b).wait()
    acc = process(buf_b, 1, acc)
    chunk_copy(3, buf_b, sem_b).start()
    chunk_copy(2, buf_a, sem_a).wait()
    acc = process(buf_a, 2, acc)
    chunk_copy(3, buf_b, sem_b).wait()
    acc = process(buf_b, 3, acc)

    acc_v[...] = acc
    pltpu.sync_copy(acc_v, out_hbm.at[wid])


def kernel(prob, label):
    partials = _iou_partials(prob.T, label)
    return jnp.sum(partials) / jnp.float32(_BATCH)


# speedup vs baseline: 2.8469x; 1.1327x over previous
import functools

import jax
import jax.numpy as jnp
from jax import lax
from jax.experimental import pallas as pl
from jax.experimental.pallas import tpu as pltpu
from jax.experimental.pallas import tpu_sc as plsc

_BATCH = 16384
_NCLS = 392
_NC = 2
_NS = 16
_NW = _NC * _NS
_L = 16
_ROWS_PER_W = _BATCH // _NW
_CHUNK = 128
_NCHUNK = _ROWS_PER_W // _CHUNK
_GPC = _CHUNK // _L

_mesh = plsc.VectorSubcoreMesh(core_axis_name="c", subcore_axis_name="s")


@functools.partial(
    pl.kernel,
    mesh=_mesh,
    out_type=jax.ShapeDtypeStruct((_NW, _L), jnp.float32),
    scratch_types=[
        pltpu.VMEM((_NCLS, _CHUNK), jnp.float32),
        pltpu.VMEM((_NCLS, _CHUNK), jnp.float32),
        pltpu.VMEM((_ROWS_PER_W,), jnp.int32),
        pltpu.VMEM((_L,), jnp.float32),
        pltpu.SemaphoreType.DMA,
        pltpu.SemaphoreType.DMA,
    ],
    compiler_params=pltpu.CompilerParams(
        use_tc_tiling_on_sc=True, needs_layout_passes=False),
)
def _iou_partials(probt_hbm, label_hbm, out_hbm,
                  buf_a, buf_b, lab_v, acc_v, sem_a, sem_b):
    wid = lax.axis_index("s") * _NC + lax.axis_index("c")
    col0 = wid * _ROWS_PER_W
    pltpu.sync_copy(label_hbm.at[pl.ds(col0, _ROWS_PER_W)], lab_v)

    iota = lax.iota(jnp.int32, _L)
    neg_inf = jnp.full((_L,), -jnp.inf, jnp.float32)

    def chunk_copy(c, buf, sem):
        return pltpu.make_async_copy(
            probt_hbm.at[:, pl.ds(col0 + c * _CHUNK, _CHUNK)], buf, sem)

    def process(buf, chunk, acc):
        def group_body(g, acc):
            gcol = g * _L

            def j_body(j, ms):
                m0, m1, m2, m3, m4, m5 = ms
                x = buf[j, pl.ds(gcol, _L)]
                n0 = jnp.maximum(m0, x)
                c = jnp.minimum(m0, x)
                n1 = jnp.maximum(m1, c)
                c = jnp.minimum(m1, c)
                n2 = jnp.maximum(m2, c)
                c = jnp.minimum(m2, c)
                n3 = jnp.maximum(m3, c)
                c = jnp.minimum(m3, c)
                n4 = jnp.maximum(m4, c)
                c = jnp.minimum(m4, c)
                n5 = jnp.maximum(m5, c)
                return (n0, n1, n2, n3, n4, n5)

            m0, m1, m2, m3, m4, m5 = lax.fori_loop(
                0, _NCLS, j_body, (neg_inf,) * 6, unroll=8)

            t = m5
            cnt = ((m0 > t).astype(jnp.float32)
                   + (m1 > t).astype(jnp.float32)
                   + (m2 > t).astype(jnp.float32)
                   + (m3 > t).astype(jnp.float32)
                   + (m4 > t).astype(jnp.float32))
            lab16 = plsc.load_gather(
                lab_v, [(chunk * _GPC + g) * _L + iota])
            labval = plsc.load_gather(buf, [lab16, gcol + iota])
            inter = (labval > t).astype(jnp.float32)
            union = cnt + 1.0 - inter
            return acc + inter / union

        return lax.fori_loop(0, _GPC, group_body, acc)

    chunk_copy(0, buf_a, sem_a).start()

    def pair_body(i, acc):
        ca = 2 * i
        cb = 2 * i + 1
        chunk_copy(ca, buf_a, sem_a).wait()
        chunk_copy(cb, buf_b, sem_b).start()
        acc = process(buf_a, ca, acc)
        chunk_copy(cb, buf_b, sem_b).wait()
        chunk_copy(jnp.minimum(ca + 2, _NCHUNK - 1), buf_a, sem_a).start()
        acc = process(buf_b, cb, acc)
        return acc

    acc = lax.fori_loop(0, _NCHUNK // 2, pair_body,
                        jnp.zeros((_L,), jnp.float32))
    chunk_copy(_NCHUNK - 1, buf_a, sem_a).wait()
    acc_v[...] = acc
    pltpu.sync_copy(acc_v, out_hbm.at[wid])


def kernel(prob, label):
    partials = _iou_partials(prob.T, label)
    return jnp.sum(partials) / jnp.float32(_BATCH)


# speedup vs baseline: 3.3578x; 1.1795x over previous
import functools

import jax
import jax.numpy as jnp
from jax import lax
from jax.experimental import pallas as pl
from jax.experimental.pallas import tpu as pltpu
from jax.experimental.pallas import tpu_sc as plsc

_BATCH = 16384
_NCLS = 392
_NC = 2
_NS = 16
_NW = _NC * _NS
_L = 16

_SC_BATCH = 8192
_ROWS_PER_W = _SC_BATCH // _NW
_CHUNK = 128
_NCHUNK = _ROWS_PER_W // _CHUNK
_GPC = _CHUNK // _L

_CB = 256
_A = _NCLS // 8
_TC_BLOCKS = (_BATCH - _SC_BATCH) // _CB
_TC_OFF = _SC_BATCH // _CB

_mesh = plsc.VectorSubcoreMesh(core_axis_name="c", subcore_axis_name="s")


@functools.partial(
    pl.kernel,
    mesh=_mesh,
    out_type=jax.ShapeDtypeStruct((_NW, _L), jnp.float32),
    scratch_types=[
        pltpu.VMEM((_NCLS, _CHUNK), jnp.float32),
        pltpu.VMEM((_NCLS, _CHUNK), jnp.float32),
        pltpu.VMEM((_ROWS_PER_W,), jnp.int32),
        pltpu.VMEM((_L,), jnp.float32),
        pltpu.SemaphoreType.DMA,
        pltpu.SemaphoreType.DMA,
    ],
    compiler_params=pltpu.CompilerParams(
        use_tc_tiling_on_sc=True, needs_layout_passes=False),
)
def _iou_partials_sc(probt_hbm, label_hbm, out_hbm,
                     buf_a, buf_b, lab_v, acc_v, sem_a, sem_b):
    wid = lax.axis_index("s") * _NC + lax.axis_index("c")
    col0 = wid * _ROWS_PER_W
    pltpu.sync_copy(label_hbm.at[pl.ds(col0, _ROWS_PER_W)], lab_v)

    iota = lax.iota(jnp.int32, _L)
    neg_inf = jnp.full((_L,), -jnp.inf, jnp.float32)

    def chunk_copy(c, buf, sem):
        return pltpu.make_async_copy(
            probt_hbm.at[:, pl.ds(col0 + c * _CHUNK, _CHUNK)], buf, sem)

    def process(buf, chunk, acc):
        def group_body(g, acc):
            gcol = g * _L

            def j_body(j, ms):
                m0, m1, m2, m3, m4, m5 = ms
                x = buf[j, pl.ds(gcol, _L)]
                n0 = jnp.maximum(m0, x)
                c = jnp.minimum(m0, x)
                n1 = jnp.maximum(m1, c)
                c = jnp.minimum(m1, c)
                n2 = jnp.maximum(m2, c)
                c = jnp.minimum(m2, c)
                n3 = jnp.maximum(m3, c)
                c = jnp.minimum(m3, c)
                n4 = jnp.maximum(m4, c)
                c = jnp.minimum(m4, c)
                n5 = jnp.maximum(m5, c)
                return (n0, n1, n2, n3, n4, n5)

            m0, m1, m2, m3, m4, m5 = lax.fori_loop(
                0, _NCLS, j_body, (neg_inf,) * 6, unroll=8)

            t = m5
            cnt = ((m0 > t).astype(jnp.float32)
                   + (m1 > t).astype(jnp.float32)
                   + (m2 > t).astype(jnp.float32)
                   + (m3 > t).astype(jnp.float32)
                   + (m4 > t).astype(jnp.float32))
            lab16 = plsc.load_gather(
                lab_v, [(chunk * _GPC + g) * _L + iota])
            labval = plsc.load_gather(buf, [lab16, gcol + iota])
            inter = (labval > t).astype(jnp.float32)
            union = cnt + 1.0 - inter
            return acc + inter / union

        return lax.fori_loop(0, _GPC, group_body, acc)

    chunk_copy(0, buf_a, sem_a).start()

    def pair_body(i, acc):
        ca = 2 * i
        cb = 2 * i + 1
        chunk_copy(ca, buf_a, sem_a).wait()
        chunk_copy(cb, buf_b, sem_b).start()
        acc = process(buf_a, ca, acc)
        chunk_copy(cb, buf_b, sem_b).wait()
        chunk_copy(jnp.minimum(ca + 2, _NCHUNK - 1), buf_a, sem_a).start()
        acc = process(buf_b, cb, acc)
        return acc

    acc = lax.fori_loop(0, _NCHUNK // 2, pair_body,
                        jnp.zeros((_L,), jnp.float32))
    chunk_copy(_NCHUNK - 1, buf_a, sem_a).wait()
    acc_v[...] = acc
    pltpu.sync_copy(acc_v, out_hbm.at[wid])


def _iou_tc_body(lab_ref, x_ref, out_ref):
    i = pl.program_id(0)
    x = x_ref[...].reshape(_A, 8, _CB)
    lab = lab_ref[0, 0, :]
    lab2 = jnp.broadcast_to(lab[None, :], (8, _CB))
    subi = lax.broadcasted_iota(jnp.int32, (8, _CB), 0)
    labrel = lab2 - subi

    neg = jnp.full((8, _CB), -jnp.inf, jnp.float32)
    m = [neg] * 6
    labacc = neg
    for a in range(_A):
        xa = x[a]
        c = xa
        n0 = jnp.maximum(m[0], c)
        c = jnp.minimum(m[0], c)
        n1 = jnp.maximum(m[1], c)
        c = jnp.minimum(m[1], c)
        n2 = jnp.maximum(m[2], c)
        c = jnp.minimum(m[2], c)
        n3 = jnp.maximum(m[3], c)
        c = jnp.minimum(m[3], c)
        n4 = jnp.maximum(m[4], c)
        c = jnp.minimum(m[4], c)
        n5 = jnp.maximum(m[5], c)
        m = [n0, n1, n2, n3, n4, n5]
        labacc = jnp.where(labrel == 8 * a, xa, labacc)

    cand = list(m)
    r = jnp.zeros((1, _CB), jnp.float32)
    t = jnp.full((1, _CB), -jnp.inf, jnp.float32)
    done = jnp.zeros((1, _CB), jnp.bool_)
    for _ in range(6):
        mm = m[0]
        for k in range(1, 6):
            mm = jnp.maximum(mm, m[k])
        mx = jnp.max(mm, axis=0, keepdims=True)
        cnt = jnp.zeros((1, _CB), jnp.float32)
        for k in range(6):
            cnt = cnt + jnp.sum((m[k] == mx).astype(jnp.float32),
                                axis=0, keepdims=True)
        new_r = r + cnt
        hit = jnp.logical_and(jnp.logical_not(done), new_r >= 6.0)
        t = jnp.where(hit, mx, t)
        done = jnp.logical_or(done, new_r >= 6.0)
        for k in range(6):
            m[k] = jnp.where(m[k] == mx, -jnp.inf, m[k])
        r = new_r

    c_pred = jnp.zeros((1, _CB), jnp.float32)
    for k in range(6):
        c_pred = c_pred + jnp.sum((cand[k] > t).astype(jnp.float32),
                                  axis=0, keepdims=True)
    labval = jnp.max(labacc, axis=0, keepdims=True)
    inter = (labval > t).astype(jnp.float32)
    union = c_pred + 1.0 - inter
    iou = inter / union

    @pl.when(i == 0)
    def _():
        out_ref[...] = jnp.zeros((1, _CB), jnp.float32)
    out_ref[...] += iou


_iou_tc = pl.pallas_call(
    _iou_tc_body,
    grid=(_TC_BLOCKS,),
    in_specs=[
        pl.BlockSpec((1, 1, _CB), lambda i: (i + _TC_OFF, 0, 0)),
        pl.BlockSpec((_NCLS, _CB), lambda i: (0, i + _TC_OFF)),
    ],
    out_specs=pl.BlockSpec((1, _CB), lambda i: (0, 0)),
    out_shape=jax.ShapeDtypeStruct((1, _CB), jnp.float32),
)


def kernel(prob, label):
    probt = prob.T
    sc_partials = _iou_partials_sc(probt, label)
    tc_partial = _iou_tc(label.reshape(_BATCH // _CB, 1, _CB), probt)
    total = jnp.sum(sc_partials) + jnp.sum(tc_partial)
    return total / jnp.float32(_BATCH)


# speedup vs baseline: 3.4173x; 1.0177x over previous
import functools

import jax
import jax.numpy as jnp
from jax import lax
from jax.experimental import pallas as pl
from jax.experimental.pallas import tpu as pltpu
from jax.experimental.pallas import tpu_sc as plsc

_BATCH = 16384
_NCLS = 392
_NC = 2
_NS = 16
_NW = _NC * _NS
_L = 16

_SC_BATCH = 8192
_ROWS_PER_W = _SC_BATCH // _NW
_CHUNK = 128
_NCHUNK = _ROWS_PER_W // _CHUNK
_GPC = _CHUNK // _L

_CB = 256
_A = _NCLS // 8
_TC_BLOCKS = (_BATCH - _SC_BATCH) // _CB
_TC_OFF = _SC_BATCH // _CB

_mesh = plsc.VectorSubcoreMesh(core_axis_name="c", subcore_axis_name="s")


@functools.partial(
    pl.kernel,
    mesh=_mesh,
    out_type=jax.ShapeDtypeStruct((_NW, _L), jnp.float32),
    scratch_types=[
        pltpu.VMEM((_NCLS, _CHUNK), jnp.float32),
        pltpu.VMEM((_NCLS, _CHUNK), jnp.float32),
        pltpu.VMEM((_ROWS_PER_W,), jnp.int32),
        pltpu.VMEM((_L,), jnp.float32),
        pltpu.SemaphoreType.DMA,
        pltpu.SemaphoreType.DMA,
    ],
    compiler_params=pltpu.CompilerParams(
        use_tc_tiling_on_sc=True, needs_layout_passes=False),
)
def _iou_partials_sc(probt_hbm, label_hbm, out_hbm,
                     buf_a, buf_b, lab_v, acc_v, sem_a, sem_b):
    wid = lax.axis_index("s") * _NC + lax.axis_index("c")
    col0 = wid * _ROWS_PER_W
    pltpu.sync_copy(label_hbm.at[pl.ds(col0, _ROWS_PER_W)], lab_v)

    iota = lax.iota(jnp.int32, _L)
    neg_inf = jnp.full((_L,), -jnp.inf, jnp.float32)

    def chunk_copy(c, buf, sem):
        return pltpu.make_async_copy(
            probt_hbm.at[:, pl.ds(col0 + c * _CHUNK, _CHUNK)], buf, sem)

    def process(buf, chunk, acc):
        def group_body(g, acc):
            gcol = g * _L

            def j_body(j, ms):
                m0, m1, m2, m3, m4, m5 = ms
                x = buf[j, pl.ds(gcol, _L)]
                n0 = jnp.maximum(m0, x)
                c = jnp.minimum(m0, x)
                n1 = jnp.maximum(m1, c)
                c = jnp.minimum(m1, c)
                n2 = jnp.maximum(m2, c)
                c = jnp.minimum(m2, c)
                n3 = jnp.maximum(m3, c)
                c = jnp.minimum(m3, c)
                n4 = jnp.maximum(m4, c)
                c = jnp.minimum(m4, c)
                n5 = jnp.maximum(m5, c)
                return (n0, n1, n2, n3, n4, n5)

            m0, m1, m2, m3, m4, m5 = lax.fori_loop(
                0, _NCLS, j_body, (neg_inf,) * 6, unroll=8)

            t = m5
            cnt = ((m0 > t).astype(jnp.float32)
                   + (m1 > t).astype(jnp.float32)
                   + (m2 > t).astype(jnp.float32)
                   + (m3 > t).astype(jnp.float32)
                   + (m4 > t).astype(jnp.float32))
            lab16 = plsc.load_gather(
                lab_v, [(chunk * _GPC + g) * _L + iota])
            labval = plsc.load_gather(buf, [lab16, gcol + iota])
            inter = (labval > t).astype(jnp.float32)
            union = cnt + 1.0 - inter
            return acc + inter / union

        return lax.fori_loop(0, _GPC, group_body, acc)

    chunk_copy(0, buf_a, sem_a).start()

    def pair_body(i, acc):
        ca = 2 * i
        cb = 2 * i + 1
        chunk_copy(ca, buf_a, sem_a).wait()
        chunk_copy(cb, buf_b, sem_b).start()
        acc = process(buf_a, ca, acc)
        chunk_copy(cb, buf_b, sem_b).wait()
        chunk_copy(jnp.minimum(ca + 2, _NCHUNK - 1), buf_a, sem_a).start()
        acc = process(buf_b, cb, acc)
        return acc

    acc = lax.fori_loop(0, _NCHUNK // 2, pair_body,
                        jnp.zeros((_L,), jnp.float32))
    chunk_copy(_NCHUNK - 1, buf_a, sem_a).wait()
    acc_v[...] = acc
    pltpu.sync_copy(acc_v, out_hbm.at[wid])


def _iou_tc_body(lab_ref, x_ref, out_ref):
    i = pl.program_id(0)
    x = x_ref[...].reshape(_A, 8, _CB)
    lab = lab_ref[0, 0, :]
    lab2 = jnp.broadcast_to(lab[None, :], (8, _CB))
    subi = lax.broadcasted_iota(jnp.int32, (8, _CB), 0)
    labrel = lab2 - subi

    neg = jnp.full((8, _CB), -jnp.inf, jnp.float32)
    m = [neg] * 6
    labacc = neg
    for a in range(_A):
        xa = x[a]
        c = xa
        n0 = jnp.maximum(m[0], c)
        c = jnp.minimum(m[0], c)
        n1 = jnp.maximum(m[1], c)
        c = jnp.minimum(m[1], c)
        n2 = jnp.maximum(m[2], c)
        c = jnp.minimum(m[2], c)
        n3 = jnp.maximum(m[3], c)
        c = jnp.minimum(m[3], c)
        n4 = jnp.maximum(m[4], c)
        c = jnp.minimum(m[4], c)
        n5 = jnp.maximum(m[5], c)
        m = [n0, n1, n2, n3, n4, n5]
        labacc = jnp.where(labrel == 8 * a, xa, labacc)

    cand = list(m)
    r = jnp.zeros((1, _CB), jnp.float32)
    t = jnp.full((1, _CB), -jnp.inf, jnp.float32)
    done = jnp.zeros((1, _CB), jnp.bool_)
    for _ in range(6):
        mm = m[0]
        for k in range(1, 6):
            mm = jnp.maximum(mm, m[k])
        mx = jnp.max(mm, axis=0, keepdims=True)
        eqs = (m[0] == mx).astype(jnp.float32)
        for k in range(1, 6):
            eqs = eqs + (m[k] == mx).astype(jnp.float32)
        new_r = r + jnp.sum(eqs, axis=0, keepdims=True)
        hit = jnp.logical_and(jnp.logical_not(done), new_r >= 6.0)
        t = jnp.where(hit, mx, t)
        done = jnp.logical_or(done, new_r >= 6.0)
        for k in range(6):
            m[k] = jnp.where(m[k] == mx, -jnp.inf, m[k])
        r = new_r

    gts = (cand[0] > t).astype(jnp.float32)
    for k in range(1, 6):
        gts = gts + (cand[k] > t).astype(jnp.float32)
    c_pred = jnp.sum(gts, axis=0, keepdims=True)
    labval = jnp.max(labacc, axis=0, keepdims=True)
    inter = (labval > t).astype(jnp.float32)
    union = c_pred + 1.0 - inter
    iou = inter / union

    @pl.when(i == 0)
    def _():
        out_ref[...] = jnp.zeros((1, _CB), jnp.float32)
    out_ref[...] += iou


_iou_tc = pl.pallas_call(
    _iou_tc_body,
    grid=(_TC_BLOCKS,),
    in_specs=[
        pl.BlockSpec((1, 1, _CB), lambda i: (i + _TC_OFF, 0, 0)),
        pl.BlockSpec((_NCLS, _CB), lambda i: (0, i + _TC_OFF)),
    ],
    out_specs=pl.BlockSpec((1, _CB), lambda i: (0, 0)),
    out_shape=jax.ShapeDtypeStruct((1, _CB), jnp.float32),
)


def kernel(prob, label):
    probt = prob.T
    sc_partials = _iou_partials_sc(probt, label)
    tc_partial = _iou_tc(label.reshape(_BATCH // _CB, 1, _CB), probt)
    total = jnp.sum(sc_partials) + jnp.sum(tc_partial)
    return total / jnp.float32(_BATCH)
